# baseline (speedup 1.0000x reference)
import functools

import jax
import jax.numpy as jnp
from jax import lax
from jax.experimental import pallas as pl
from jax.experimental.pallas import tpu as pltpu
from jax.experimental.pallas import tpu_sc as plsc

BATCH = 16384
HIST = 50
D = 16
N_FLAT = BATCH * HIST

NC = 2
NS = 16
NW = NC * NS
B_PER_W = N_FLAT // NW
CHUNK = 3200
N_CHUNKS = B_PER_W // CHUNK

_mesh = plsc.VectorSubcoreMesh(core_axis_name="c", subcore_axis_name="s")


@functools.partial(
    pl.kernel,
    mesh=_mesh,
    out_type=jax.ShapeDtypeStruct((N_FLAT, D), jnp.float32),
    scratch_types=[
        pltpu.VMEM((CHUNK,), jnp.int32),
        pltpu.VMEM((CHUNK, D), jnp.float32),
        pltpu.SemaphoreType.DMA,
    ],
    compiler_params=pltpu.CompilerParams(use_tc_tiling_on_sc=False),
)
def _gather_kernel(idx_hbm, table_hbm, out_hbm, idx_v, rows_v, sem):
    wid = lax.axis_index("s") * NC + lax.axis_index("c")
    base = pl.multiple_of(wid * B_PER_W, B_PER_W)
    for g in range(N_CHUNKS):
        off = base + g * CHUNK
        pltpu.sync_copy(idx_hbm.at[pl.ds(off, CHUNK)], idx_v)
        pltpu.async_copy(table_hbm.at[idx_v], rows_v, sem).wait()
        pltpu.sync_copy(rows_v, out_hbm.at[pl.ds(off, CHUNK)])


def kernel(x, table):
    idx = x.reshape(N_FLAT).astype(jnp.int32)
    out = _gather_kernel(idx, table)
    return out.reshape(BATCH, HIST, D)


# speedup vs baseline: 1.0108x; 1.0108x over previous
import functools

import jax
import jax.numpy as jnp
from jax import lax
from jax.experimental import pallas as pl
from jax.experimental.pallas import tpu as pltpu
from jax.experimental.pallas import tpu_sc as plsc

BATCH = 16384
HIST = 50
D = 16
N_FLAT = BATCH * HIST

NC = 2
NS = 16
NW = NC * NS
B_PER_W = N_FLAT // NW
CHUNK = 3200
N_CHUNKS = B_PER_W // CHUNK

_mesh = plsc.VectorSubcoreMesh(core_axis_name="c", subcore_axis_name="s")


@functools.partial(
    pl.kernel,
    mesh=_mesh,
    out_type=jax.ShapeDtypeStruct((N_FLAT, D), jnp.float32),
    scratch_types=[
        pltpu.VMEM((B_PER_W,), jnp.int32),
        pltpu.VMEM((CHUNK, D), jnp.float32),
        pltpu.VMEM((CHUNK, D), jnp.float32),
        pltpu.SemaphoreType.DMA,
        pltpu.SemaphoreType.DMA,
        pltpu.SemaphoreType.DMA,
        pltpu.SemaphoreType.DMA,
    ],
    compiler_params=pltpu.CompilerParams(use_tc_tiling_on_sc=False),
)
def _gather_kernel(idx_hbm, table_hbm, out_hbm, idx_v, rows0, rows1,
                   gs0, gs1, os0, os1):
    wid = lax.axis_index("s") * NC + lax.axis_index("c")
    base = pl.multiple_of(wid * B_PER_W, B_PER_W)
    pltpu.sync_copy(idx_hbm.at[pl.ds(base, B_PER_W)], idx_v)

    rows = (rows0, rows1)
    gsem = (gs0, gs1)
    osem = (os0, os1)

    def start_gather(g):
        return pltpu.async_copy(
            table_hbm.at[idx_v.at[pl.ds(g * CHUNK, CHUNK)]],
            rows[g % 2], gsem[g % 2])

    gathers = [None] * N_CHUNKS
    outs = [None] * N_CHUNKS
    gathers[0] = start_gather(0)
    for g in range(N_CHUNKS):
        if g + 1 < N_CHUNKS:
            if g >= 1:
                outs[g - 1].wait()
            gathers[g + 1] = start_gather(g + 1)
        gathers[g].wait()
        outs[g] = pltpu.async_copy(
            rows[g % 2], out_hbm.at[pl.ds(base + g * CHUNK, CHUNK)],
            osem[g % 2])
    outs[N_CHUNKS - 2].wait()
    outs[N_CHUNKS - 1].wait()


def kernel(x, table):
    idx = x.reshape(N_FLAT).astype(jnp.int32)
    out = _gather_kernel(idx, table)
    return out.reshape(BATCH, HIST, D)


# speedup vs baseline: 1.3538x; 1.3393x over previous
import functools

import jax
import jax.numpy as jnp
from jax import lax
from jax.experimental import pallas as pl
from jax.experimental.pallas import tpu as pltpu
from jax.experimental.pallas import tpu_sc as plsc

BATCH = 16384
HIST = 50
D = 16
N_FLAT = BATCH * HIST

NC = 2
NS = 16
NW = NC * NS
B_PER_W = N_FLAT // NW
CHUNK = 3200
N_CHUNKS = B_PER_W // CHUNK

_mesh = plsc.VectorSubcoreMesh(core_axis_name="c", subcore_axis_name="s")


@functools.partial(
    pl.kernel,
    mesh=_mesh,
    out_type=jax.ShapeDtypeStruct((N_FLAT, D), jnp.float32),
    scratch_types=[
        pltpu.VMEM((B_PER_W,), jnp.int32),
        pltpu.VMEM((CHUNK, D), jnp.float32),
        pltpu.VMEM((CHUNK, D), jnp.float32),
        pltpu.SemaphoreType.DMA,
        pltpu.SemaphoreType.DMA,
        pltpu.SemaphoreType.DMA,
        pltpu.SemaphoreType.DMA,
    ],
    compiler_params=pltpu.CompilerParams(use_tc_tiling_on_sc=False),
)
def _gather_kernel(idx_hbm, table_hbm, out_hbm, idx_v, rows0, rows1,
                   gs0, gs1, os0, os1):
    wid = lax.axis_index("s") * NC + lax.axis_index("c")
    base = pl.multiple_of(wid * B_PER_W, B_PER_W)
    pltpu.sync_copy(idx_hbm.at[pl.ds(base, B_PER_W)], idx_v)

    rows = (rows0, rows1)
    gsem = (gs0, gs1)
    osem = (os0, os1)

    def start_gather(g):
        return pltpu.async_copy(
            table_hbm.at[idx_v.at[pl.ds(g * CHUNK, CHUNK)]],
            rows[g % 2], gsem[g % 2])

    gathers = [None] * N_CHUNKS
    outs = [None] * N_CHUNKS
    gathers[0] = start_gather(0)
    for g in range(N_CHUNKS):
        if g + 1 < N_CHUNKS:
            if g >= 1:
                outs[g - 1].wait()
            gathers[g + 1] = start_gather(g + 1)
        gathers[g].wait()
        outs[g] = pltpu.async_copy(
            rows[g % 2], out_hbm.at[pl.ds(base + g * CHUNK, CHUNK)],
            osem[g % 2])
    outs[N_CHUNKS - 2].wait()
    outs[N_CHUNKS - 1].wait()


def _tc_transpose_body(i_ref, o_ref):
    o_ref[...] = i_ref[...].T


_tc_transpose = pl.pallas_call(
    _tc_transpose_body,
    grid=(HIST,),
    in_specs=[pl.BlockSpec((BATCH, D), lambda i: (i, 0))],
    out_specs=pl.BlockSpec((D, BATCH), lambda i: (i, 0)),
    out_shape=jax.ShapeDtypeStruct((HIST * D, BATCH), jnp.float32),
)


def kernel(x, table):
    idx = x.T.reshape(N_FLAT).astype(jnp.int32)
    rows = _gather_kernel(idx, table)
    out2 = _tc_transpose(rows)
    return out2.reshape(HIST, D, BATCH).transpose(2, 0, 1)
